# baseline (speedup 1.0000x reference)
import functools

import jax
import jax.numpy as jnp
from jax.experimental import pallas as pl
from jax.experimental.pallas import tpu as pltpu
from jax.experimental.pallas import tpu_sc as plsc

_K = 32
_TAIL = 128
_CAP = 512


def _score_kernel(q_ref, k_ref, s_ref):
    q = q_ref[0]
    k = k_ref[0]
    s = jax.lax.dot_general(q, k, (((1,), (1,)), ((), ())),
                            preferred_element_type=jnp.float32)
    sq, n_kv = s.shape
    neg = jnp.float32(-jnp.inf)
    m = jnp.max(s.reshape(sq, n_kv // 128, 128), axis=2)

    def body(t, carry):
        m, _ = carry
        mu = jnp.max(m, axis=1, keepdims=True)
        return jnp.where(m == mu, neg, m), mu

    _, tau = jax.lax.fori_loop(0, _K, body, (m, jnp.zeros((sq, 1), jnp.float32)))
    s_ref[0] = jnp.concatenate(
        [s, jnp.broadcast_to(tau, (sq, _TAIL))], axis=1)


def _scores_tau(Qf, Kf):
    G, Sq, D = Qf.shape
    Skv = Kf.shape[1]
    return pl.pallas_call(
        _score_kernel,
        grid=(G,),
        in_specs=[pl.BlockSpec((1, Sq, D), lambda g: (g, 0, 0)),
                  pl.BlockSpec((1, Skv, D), lambda g: (g, 0, 0))],
        out_specs=pl.BlockSpec((1, Sq, Skv + _TAIL), lambda g: (g, 0, 0)),
        out_shape=jax.ShapeDtypeStruct((G, Sq, Skv + _TAIL), jnp.float32),
    )(Qf, Kf)


def _sc_topk_attention(G, Skv):
    hp = G // 32
    hd = _K * _K
    rw = Skv + _TAIL
    vstride = Skv * _K
    neg = jnp.float32(-jnp.inf)
    big = jnp.int32(1 << 30)

    mesh = plsc.VectorSubcoreMesh(core_axis_name="c", subcore_axis_name="s")

    @functools.partial(
        pl.kernel, mesh=mesh,
        compiler_params=pltpu.CompilerParams(needs_layout_passes=False),
        out_type=jax.ShapeDtypeStruct((G * hd,), jnp.float32),
        scratch_types=[
            pltpu.VMEM((8 * rw,), jnp.float32),
            pltpu.VMEM((_CAP,), jnp.float32),
            pltpu.VMEM((_CAP,), jnp.int32),
            pltpu.VMEM((hd,), jnp.int32),
            pltpu.VMEM((8, 128), jnp.int32),
            pltpu.VMEM((8, 128), jnp.float32),
            pltpu.VMEM((hd,), jnp.float32),
            pltpu.VMEM((hd,), jnp.float32),
            pltpu.SemaphoreType.DMA,
        ],
    )
    def sc_kernel(s_hbm, v_hbm, out_hbm,
                  rbuf, cv, ci, idx_v, fidx_v, tv_v, p_v, out_v, sem):
        wid = jax.lax.axis_index("s") * 2 + jax.lax.axis_index("c")
        iota = jax.lax.iota(jnp.int32, 16)
        zeros16 = jnp.zeros((16, 1), jnp.int32)
        dnums = jax.lax.GatherDimensionNumbers(
            offset_dims=(), collapsed_slice_dims=(0,), start_index_map=(0,))

        def bcast0(vec):
            return jax.lax.gather(
                vec, zeros16, dnums, (1,),
                mode=jax.lax.GatherScatterMode.PROMISE_IN_BOUNDS)

        def head_body(h, carry):
            head = wid * hp + h
            base = head * hd

            def block_body(b, carry):
                pltpu.sync_copy(
                    s_hbm.at[pl.ds((head * _K + b * 8) * rw, 8 * rw)], rbuf)

                def row_body(r, carry):
                    rb = r * rw
                    j = b * 8 + r
                    tau16 = rbuf[pl.ds(rb + Skv, 16)]
                    def pre_body(u, c):
                        cv[pl.ds(u * 16, 16)] = jnp.full((16,), neg)
                        return c
                    jax.lax.fori_loop(0, _CAP // 16, pre_body, 0)

                    def grp_body(g, off):
                        gb = rb + g * 128
                        ms = [rbuf[pl.ds(gb + v * 16, 16)] >= tau16
                              for v in range(8)]
                        mo = ms[0]
                        for v in range(1, 8):
                            mo = jnp.logical_or(mo, ms[v])

                        def collect(off):
                            for v in range(8):
                                sv = rbuf[pl.ds(gb + v * 16, 16)]
                                mv = sv >= tau16
                                cums = plsc.cumsum(mv.astype(jnp.int32))
                                pos = jnp.minimum(off + cums - 1, _CAP - 1)
                                ixv = iota + (g * 128 + v * 16)
                                plsc.store_scatter(cv, [pos], sv, mask=mv)
                                plsc.store_scatter(ci, [pos], ixv, mask=mv)
                                off = off + jnp.max(cums)
                            return off

                        return jax.lax.cond(jnp.any(mo), collect,
                                            lambda off: off, off)

                    off = jax.lax.fori_loop(0, Skv // 128, grp_body,
                                            jnp.int32(0))
                    nv = jnp.minimum((off + 15) // 16, _CAP // 16)

                    def ext_body(t, carry):
                        lo_v, hi_v, lo_i, hi_i = carry

                        def p1(u, bc):
                            bv, bi = bc
                            v = cv[pl.ds(u * 16, 16)]
                            ix = ci[pl.ds(u * 16, 16)]
                            bet = (v > bv) | ((v == bv) & (ix < bi))
                            return (jnp.where(bet, v, bv),
                                    jnp.where(bet, ix, bi))

                        bv, bi = jax.lax.fori_loop(
                            0, nv, p1,
                            (jnp.full((16,), neg), jnp.full((16,), big)))
                        mu = jnp.max(bv)
                        imin = jnp.min(jnp.where(bv == mu, bi, big))

                        def p2(u, c):
                            v = cv[pl.ds(u * 16, 16)]
                            ix = ci[pl.ds(u * 16, 16)]
                            kill = (v == mu) & (ix == imin)
                            cv[pl.ds(u * 16, 16)] = jnp.where(kill, neg, v)
                            return c
                        jax.lax.fori_loop(0, nv, p2, 0)

                        tl = iota == t
                        th = iota == (t - 16)
                        return (jnp.where(tl, mu, lo_v),
                                jnp.where(th, mu, hi_v),
                                jnp.where(tl, imin, lo_i),
                                jnp.where(th, imin, hi_i))

                    z16f = jnp.zeros((16,), jnp.float32)
                    z16i = jnp.zeros((16,), jnp.int32)
                    lo_v, hi_v, lo_i, hi_i = jax.lax.fori_loop(
                        0, _K, ext_body, (z16f, z16f, z16i, z16i))

                    mx = bcast0(lo_v)
                    el = jnp.exp(lo_v - mx)
                    eh = jnp.exp(hi_v - mx)
                    den = jnp.zeros((16,), jnp.float32) + (jnp.sum(el) +
                                                           jnp.sum(eh))
                    p_v[pl.ds(j * _K, 16)] = el / den
                    p_v[pl.ds(j * _K + 16, 16)] = eh / den
                    idx_v[pl.ds(j * _K, 16)] = lo_i
                    idx_v[pl.ds(j * _K + 16, 16)] = hi_i
                    return carry

                jax.lax.fori_loop(0, 8, row_body, 0)
                return carry

            jax.lax.fori_loop(0, 4, block_body, 0)

            voff = head * vstride
            for v in range(64):
                r, c0 = v // 8, (v % 8) * 16
                n0 = (v % 2) * 16
                sl = idx_v[pl.ds(v * 16, 16)]
                fidx_v[r, pl.ds(c0, 16)] = sl * _K + (iota + (n0 + voff))
            cps = [pltpu.async_copy(v_hbm.at[fidx_v.at[r]], tv_v.at[r], sem)
                   for r in range(8)]
            for cp in cps:
                cp.wait()

            def row_mm(i, carry):
                acc0 = jnp.zeros((16,), jnp.float32)
                acc1 = jnp.zeros((16,), jnp.float32)
                for gq in range(2):
                    pvec = p_v[pl.ds(i * _K + gq * 16, 16)]
                    for jj in range(16):
                        jr = gq * 16 + jj
                        pj = jax.lax.gather(
                            pvec, jnp.full((16, 1), jj, jnp.int32), dnums,
                            (1,),
                            mode=jax.lax.GatherScatterMode.PROMISE_IN_BOUNDS)
                        t0 = tv_v[jr // 4, pl.ds((jr % 4) * 32, 16)]
                        t1 = tv_v[jr // 4, pl.ds((jr % 4) * 32 + 16, 16)]
                        acc0 = acc0 + pj * t0
                        acc1 = acc1 + pj * t1
                out_v[pl.ds(i * _K, 16)] = acc0
                out_v[pl.ds(i * _K + 16, 16)] = acc1
                return carry
            jax.lax.fori_loop(0, _K, row_mm, 0)
            pltpu.sync_copy(out_v, out_hbm.at[pl.ds(base, hd)])
            return carry

        jax.lax.fori_loop(0, hp, head_body, 0)

    return sc_kernel


def kernel(Q, K, V):
    B, H, Sq, D = Q.shape
    Skv = K.shape[2]
    G = B * H
    Qf = Q.reshape(G, Sq, D)
    Kf = K.reshape(G, Skv, D)
    st = _scores_tau(Qf, Kf)
    sc = _sc_topk_attention(G, Skv)
    out = sc(st.reshape(G * Sq * (Skv + _TAIL)),
             V[..., :_K].reshape(G * Skv * _K))
    return out.reshape(B, H, Sq, _K)


# speedup vs baseline: 1.5589x; 1.5589x over previous
import functools

import jax
import jax.numpy as jnp
from jax.experimental import pallas as pl
from jax.experimental.pallas import tpu as pltpu
from jax.experimental.pallas import tpu_sc as plsc

_K = 32
_TAIL = 128
_CAP = 1024


def _score_kernel(q_ref, k_ref, s_ref):
    q = q_ref[0]
    k = k_ref[0]
    s = jax.lax.dot_general(q, k, (((1,), (1,)), ((), ())),
                            preferred_element_type=jnp.float32)
    sq, n_kv = s.shape
    m = jnp.max(s.reshape(sq, _K, n_kv // _K), axis=2)
    tau = jnp.min(m, axis=1, keepdims=True)
    s_ref[0] = jnp.concatenate(
        [s, jnp.broadcast_to(tau, (sq, _TAIL))], axis=1)


def _scores_tau(Qf, Kf):
    G, Sq, D = Qf.shape
    Skv = Kf.shape[1]
    return pl.pallas_call(
        _score_kernel,
        grid=(G,),
        in_specs=[pl.BlockSpec((1, Sq, D), lambda g: (g, 0, 0)),
                  pl.BlockSpec((1, Skv, D), lambda g: (g, 0, 0))],
        out_specs=pl.BlockSpec((1, Sq, Skv + _TAIL), lambda g: (g, 0, 0)),
        out_shape=jax.ShapeDtypeStruct((G, Sq, Skv + _TAIL), jnp.float32),
    )(Qf, Kf)


def _sc_topk_attention(G, Skv):
    hp = G // 32
    hd = _K * _K
    rw = Skv + _TAIL
    vstride = Skv * _K
    neg = jnp.float32(-jnp.inf)
    big = jnp.int32(1 << 30)

    mesh = plsc.VectorSubcoreMesh(core_axis_name="c", subcore_axis_name="s")

    @functools.partial(
        pl.kernel, mesh=mesh,
        compiler_params=pltpu.CompilerParams(needs_layout_passes=False),
        out_type=jax.ShapeDtypeStruct((G * hd,), jnp.float32),
        scratch_types=[
            pltpu.VMEM((8 * rw,), jnp.float32),
            pltpu.VMEM((_CAP,), jnp.float32),
            pltpu.VMEM((_CAP,), jnp.int32),
            pltpu.VMEM((hd,), jnp.int32),
            pltpu.VMEM((8, 128), jnp.int32),
            pltpu.VMEM((8, 128), jnp.float32),
            pltpu.VMEM((hd,), jnp.float32),
            pltpu.VMEM((hd,), jnp.float32),
            pltpu.SemaphoreType.DMA,
        ],
    )
    def sc_kernel(s_hbm, v_hbm, out_hbm,
                  rbuf, cv, ci, idx_v, fidx_v, tv_v, p_v, out_v, sem):
        wid = jax.lax.axis_index("s") * 2 + jax.lax.axis_index("c")
        iota = jax.lax.iota(jnp.int32, 16)
        zeros16 = jnp.zeros((16, 1), jnp.int32)
        dnums = jax.lax.GatherDimensionNumbers(
            offset_dims=(), collapsed_slice_dims=(0,), start_index_map=(0,))

        def bcast0(vec):
            return jax.lax.gather(
                vec, zeros16, dnums, (1,),
                mode=jax.lax.GatherScatterMode.PROMISE_IN_BOUNDS)

        def head_body(h, carry):
            head = wid * hp + h
            base = head * hd

            def block_body(b, carry):
                pltpu.sync_copy(
                    s_hbm.at[pl.ds((head * _K + b * 8) * rw, 8 * rw)], rbuf)

                def row_body(r, carry):
                    rb = r * rw
                    j = b * 8 + r
                    tau16 = rbuf[pl.ds(rb + Skv, 16)]
                    def pre_body(u, c):
                        cv[pl.ds(u * 16, 16)] = jnp.full((16,), neg)
                        return c
                    jax.lax.fori_loop(0, _CAP // 16, pre_body, 0)

                    def grp_body(g, off):
                        gb = rb + g * 128
                        ms = [rbuf[pl.ds(gb + v * 16, 16)] >= tau16
                              for v in range(8)]
                        mo = ms[0]
                        for v in range(1, 8):
                            mo = jnp.logical_or(mo, ms[v])

                        def collect(off):
                            for v in range(8):
                                sv = rbuf[pl.ds(gb + v * 16, 16)]
                                mv = sv >= tau16
                                cums = plsc.cumsum(mv.astype(jnp.int32))
                                pos = jnp.minimum(off + cums - 1, _CAP - 1)
                                ixv = iota + (g * 128 + v * 16)
                                plsc.store_scatter(cv, [pos], sv, mask=mv)
                                plsc.store_scatter(ci, [pos], ixv, mask=mv)
                                off = off + jnp.max(cums)
                            return off

                        return jax.lax.cond(jnp.any(mo), collect,
                                            lambda off: off, off)

                    off = jax.lax.fori_loop(0, Skv // 128, grp_body,
                                            jnp.int32(0))
                    nv = jnp.minimum((off + 15) // 16, _CAP // 16)

                    def ext_body(t, carry):
                        lo_v, hi_v, lo_i, hi_i = carry

                        def p1(u, bc):
                            bv, bi = bc
                            v = cv[pl.ds(u * 16, 16)]
                            ix = ci[pl.ds(u * 16, 16)]
                            bet = (v > bv) | ((v == bv) & (ix < bi))
                            return (jnp.where(bet, v, bv),
                                    jnp.where(bet, ix, bi))

                        bv, bi = jax.lax.fori_loop(
                            0, nv, p1,
                            (jnp.full((16,), neg), jnp.full((16,), big)))
                        mu = jnp.max(bv)
                        imin = jnp.min(jnp.where(bv == mu, bi, big))

                        def p2(u, c):
                            v = cv[pl.ds(u * 16, 16)]
                            ix = ci[pl.ds(u * 16, 16)]
                            kill = (v == mu) & (ix == imin)
                            cv[pl.ds(u * 16, 16)] = jnp.where(kill, neg, v)
                            return c
                        jax.lax.fori_loop(0, nv, p2, 0)

                        tl = iota == t
                        th = iota == (t - 16)
                        return (jnp.where(tl, mu, lo_v),
                                jnp.where(th, mu, hi_v),
                                jnp.where(tl, imin, lo_i),
                                jnp.where(th, imin, hi_i))

                    z16f = jnp.zeros((16,), jnp.float32)
                    z16i = jnp.zeros((16,), jnp.int32)
                    lo_v, hi_v, lo_i, hi_i = jax.lax.fori_loop(
                        0, _K, ext_body, (z16f, z16f, z16i, z16i))

                    mx = bcast0(lo_v)
                    el = jnp.exp(lo_v - mx)
                    eh = jnp.exp(hi_v - mx)
                    den = jnp.zeros((16,), jnp.float32) + (jnp.sum(el) +
                                                           jnp.sum(eh))
                    p_v[pl.ds(j * _K, 16)] = el / den
                    p_v[pl.ds(j * _K + 16, 16)] = eh / den
                    idx_v[pl.ds(j * _K, 16)] = lo_i
                    idx_v[pl.ds(j * _K + 16, 16)] = hi_i
                    return carry

                jax.lax.fori_loop(0, 8, row_body, 0)
                return carry

            jax.lax.fori_loop(0, 4, block_body, 0)

            voff = head * vstride
            for v in range(64):
                r, c0 = v // 8, (v % 8) * 16
                n0 = (v % 2) * 16
                sl = idx_v[pl.ds(v * 16, 16)]
                fidx_v[r, pl.ds(c0, 16)] = sl * _K + (iota + (n0 + voff))
            cps = [pltpu.async_copy(v_hbm.at[fidx_v.at[r]], tv_v.at[r], sem)
                   for r in range(8)]
            for cp in cps:
                cp.wait()

            def row_mm(i, carry):
                acc0 = jnp.zeros((16,), jnp.float32)
                acc1 = jnp.zeros((16,), jnp.float32)
                for gq in range(2):
                    pvec = p_v[pl.ds(i * _K + gq * 16, 16)]
                    for jj in range(16):
                        jr = gq * 16 + jj
                        pj = jax.lax.gather(
                            pvec, jnp.full((16, 1), jj, jnp.int32), dnums,
                            (1,),
                            mode=jax.lax.GatherScatterMode.PROMISE_IN_BOUNDS)
                        t0 = tv_v[jr // 4, pl.ds((jr % 4) * 32, 16)]
                        t1 = tv_v[jr // 4, pl.ds((jr % 4) * 32 + 16, 16)]
                        acc0 = acc0 + pj * t0
                        acc1 = acc1 + pj * t1
                out_v[pl.ds(i * _K, 16)] = acc0
                out_v[pl.ds(i * _K + 16, 16)] = acc1
                return carry
            jax.lax.fori_loop(0, _K, row_mm, 0)
            pltpu.sync_copy(out_v, out_hbm.at[pl.ds(base, hd)])
            return carry

        jax.lax.fori_loop(0, hp, head_body, 0)

    return sc_kernel


def kernel(Q, K, V):
    B, H, Sq, D = Q.shape
    Skv = K.shape[2]
    G = B * H
    Qf = Q.reshape(G, Sq, D)
    Kf = K.reshape(G, Skv, D)
    st = _scores_tau(Qf, Kf)
    sc = _sc_topk_attention(G, Skv)
    out = sc(st.reshape(G * Sq * (Skv + _TAIL)),
             V[..., :_K].reshape(G * Skv * _K))
    return out.reshape(B, H, Sq, _K)


# speedup vs baseline: 2.1881x; 1.4036x over previous
import functools

import jax
import jax.numpy as jnp
from jax.experimental import pallas as pl
from jax.experimental.pallas import tpu as pltpu
from jax.experimental.pallas import tpu_sc as plsc

_K = 32
_TAIL = 128
_CAP = 1024


def _score_kernel(q_ref, k_ref, s_ref):
    q = q_ref[0]
    k = k_ref[0]
    s = jax.lax.dot_general(q, k, (((1,), (1,)), ((), ())),
                            preferred_element_type=jnp.float32)
    sq, n_kv = s.shape
    m = jnp.max(s.reshape(sq, _K, n_kv // _K), axis=2)
    tau = jnp.min(m, axis=1, keepdims=True)
    s_ref[0] = jnp.concatenate(
        [s, jnp.broadcast_to(tau, (sq, _TAIL))], axis=1)


def _scores_tau(Qf, Kf):
    G, Sq, D = Qf.shape
    Skv = Kf.shape[1]
    return pl.pallas_call(
        _score_kernel,
        grid=(G,),
        in_specs=[pl.BlockSpec((1, Sq, D), lambda g: (g, 0, 0)),
                  pl.BlockSpec((1, Skv, D), lambda g: (g, 0, 0))],
        out_specs=pl.BlockSpec((1, Sq, Skv + _TAIL), lambda g: (g, 0, 0)),
        out_shape=jax.ShapeDtypeStruct((G, Sq, Skv + _TAIL), jnp.float32),
    )(Qf, Kf)


def _sc_topk_attention(G, Skv):
    hp = G // 32
    hd = _K * _K
    rw = Skv + _TAIL
    vstride = Skv * _K
    neg = jnp.float32(-jnp.inf)
    big = jnp.int32(1 << 30)

    mesh = plsc.VectorSubcoreMesh(core_axis_name="c", subcore_axis_name="s")

    @functools.partial(
        pl.kernel, mesh=mesh,
        compiler_params=pltpu.CompilerParams(needs_layout_passes=False),
        out_type=jax.ShapeDtypeStruct((G * hd,), jnp.float32),
        scratch_types=[
            pltpu.VMEM((8 * rw,), jnp.float32),
            pltpu.VMEM((_CAP,), jnp.float32),
            pltpu.VMEM((_CAP,), jnp.int32),
            pltpu.VMEM((hd,), jnp.int32),
            pltpu.VMEM((8, 128), jnp.int32),
            pltpu.VMEM((8, 128), jnp.float32),
            pltpu.VMEM((hd,), jnp.float32),
            pltpu.VMEM((hd,), jnp.float32),
            pltpu.SemaphoreType.DMA,
        ],
    )
    def sc_kernel(s_hbm, v_hbm, out_hbm,
                  rbuf, cv, ci, idx_v, fidx_v, tv_v, p_v, out_v, sem):
        wid = jax.lax.axis_index("s") * 2 + jax.lax.axis_index("c")
        iota = jax.lax.iota(jnp.int32, 16)
        zeros16 = jnp.zeros((16, 1), jnp.int32)
        dnums = jax.lax.GatherDimensionNumbers(
            offset_dims=(), collapsed_slice_dims=(0,), start_index_map=(0,))

        def bcast0(vec):
            return jax.lax.gather(
                vec, zeros16, dnums, (1,),
                mode=jax.lax.GatherScatterMode.PROMISE_IN_BOUNDS)

        fifteen16 = jnp.full((16, 1), 15, jnp.int32)

        def bcast15(vec):
            return jax.lax.gather(
                vec, fifteen16, dnums, (1,),
                mode=jax.lax.GatherScatterMode.PROMISE_IN_BOUNDS)

        def head_body(h, carry):
            head = wid * hp + h
            base = head * hd

            def block_body(b, carry):
                pltpu.sync_copy(
                    s_hbm.at[pl.ds((head * _K + b * 8) * rw, 8 * rw)], rbuf)

                def row_body(r, carry):
                    rb = r * rw
                    j = b * 8 + r
                    tau16 = rbuf[pl.ds(rb + Skv, 16)]

                    def grp_body(g, offv):
                        gb = rb + g * 128
                        svs = [rbuf[pl.ds(gb + v * 16, 16)]
                               for v in range(8)]
                        ms = [sv >= tau16 for sv in svs]
                        mo = ms[0]
                        for v in range(1, 8):
                            mo = jnp.logical_or(mo, ms[v])

                        def collect(offv):
                            cums = [plsc.cumsum(m.astype(jnp.int32))
                                    for m in ms]
                            for v in range(8):
                                pos = jnp.minimum(offv + cums[v] - 1,
                                                  _CAP - 1)
                                ixv = iota + (g * 128 + v * 16)
                                plsc.store_scatter(cv, [pos], svs[v],
                                                   mask=ms[v])
                                plsc.store_scatter(ci, [pos], ixv,
                                                   mask=ms[v])
                                offv = offv + bcast15(cums[v])
                            return offv

                        return jax.lax.cond(jnp.any(mo), collect,
                                            lambda o: o, offv)

                    offv = jax.lax.fori_loop(0, Skv // 128, grp_body,
                                             jnp.zeros((16,), jnp.int32))
                    plsc.store_scatter(
                        cv, [jnp.minimum(offv + iota, _CAP - 1)],
                        jnp.full((16,), neg))
                    off = jnp.max(offv)
                    nv = jnp.minimum((off + 15) // 16, _CAP // 16)

                    def ext_body(t, carry):
                        lo_v, hi_v, lo_i, hi_i = carry

                        def p1(u, bc):
                            bv, bi = bc
                            v = cv[pl.ds(u * 16, 16)]
                            ix = ci[pl.ds(u * 16, 16)]
                            bet = (v > bv) | ((v == bv) & (ix < bi))
                            return (jnp.where(bet, v, bv),
                                    jnp.where(bet, ix, bi))

                        bv, bi = jax.lax.fori_loop(
                            0, nv, p1,
                            (jnp.full((16,), neg), jnp.full((16,), big)))
                        mu = jnp.max(bv)
                        imin = jnp.min(jnp.where(bv == mu, bi, big))

                        def p2(u, c):
                            v = cv[pl.ds(u * 16, 16)]
                            ix = ci[pl.ds(u * 16, 16)]
                            kill = (v == mu) & (ix == imin)
                            cv[pl.ds(u * 16, 16)] = jnp.where(kill, neg, v)
                            return c
                        jax.lax.fori_loop(0, nv, p2, 0)

                        tl = iota == t
                        th = iota == (t - 16)
                        return (jnp.where(tl, mu, lo_v),
                                jnp.where(th, mu, hi_v),
                                jnp.where(tl, imin, lo_i),
                                jnp.where(th, imin, hi_i))

                    z16f = jnp.zeros((16,), jnp.float32)
                    z16i = jnp.zeros((16,), jnp.int32)
                    lo_v, hi_v, lo_i, hi_i = jax.lax.fori_loop(
                        0, _K, ext_body, (z16f, z16f, z16i, z16i))

                    mx = bcast0(lo_v)
                    el = jnp.exp(lo_v - mx)
                    eh = jnp.exp(hi_v - mx)
                    den = jnp.zeros((16,), jnp.float32) + (jnp.sum(el) +
                                                           jnp.sum(eh))
                    p_v[pl.ds(j * _K, 16)] = el / den
                    p_v[pl.ds(j * _K + 16, 16)] = eh / den
                    idx_v[pl.ds(j * _K, 16)] = lo_i
                    idx_v[pl.ds(j * _K + 16, 16)] = hi_i
                    return carry

                jax.lax.fori_loop(0, 8, row_body, 0)
                return carry

            jax.lax.fori_loop(0, 4, block_body, 0)

            voff = head * vstride
            for v in range(64):
                r, c0 = v // 8, (v % 8) * 16
                n0 = (v % 2) * 16
                sl = idx_v[pl.ds(v * 16, 16)]
                fidx_v[r, pl.ds(c0, 16)] = sl * _K + (iota + (n0 + voff))
            cps = [pltpu.async_copy(v_hbm.at[fidx_v.at[r]], tv_v.at[r], sem)
                   for r in range(8)]
            for cp in cps:
                cp.wait()

            def row_mm(i, carry):
                acc0 = jnp.zeros((16,), jnp.float32)
                acc1 = jnp.zeros((16,), jnp.float32)
                for gq in range(2):
                    pvec = p_v[pl.ds(i * _K + gq * 16, 16)]
                    for jj in range(16):
                        jr = gq * 16 + jj
                        pj = jax.lax.gather(
                            pvec, jnp.full((16, 1), jj, jnp.int32), dnums,
                            (1,),
                            mode=jax.lax.GatherScatterMode.PROMISE_IN_BOUNDS)
                        t0 = tv_v[jr // 4, pl.ds((jr % 4) * 32, 16)]
                        t1 = tv_v[jr // 4, pl.ds((jr % 4) * 32 + 16, 16)]
                        acc0 = acc0 + pj * t0
                        acc1 = acc1 + pj * t1
                out_v[pl.ds(i * _K, 16)] = acc0
                out_v[pl.ds(i * _K + 16, 16)] = acc1
                return carry
            jax.lax.fori_loop(0, _K, row_mm, 0)
            pltpu.sync_copy(out_v, out_hbm.at[pl.ds(base, hd)])
            return carry

        jax.lax.fori_loop(0, hp, head_body, 0)

    return sc_kernel


def kernel(Q, K, V):
    B, H, Sq, D = Q.shape
    Skv = K.shape[2]
    G = B * H
    Qf = Q.reshape(G, Sq, D)
    Kf = K.reshape(G, Skv, D)
    st = _scores_tau(Qf, Kf)
    sc = _sc_topk_attention(G, Skv)
    out = sc(st.reshape(G * Sq * (Skv + _TAIL)),
             V[..., :_K].reshape(G * Skv * _K))
    return out.reshape(B, H, Sq, _K)
